# baseline (speedup 1.0000x reference)
import functools

import jax
import jax.numpy as jnp
from jax import lax
from jax.experimental import pallas as pl
from jax.experimental.pallas import tpu as pltpu
from jax.experimental.pallas import tpu_sc as plsc

VOCAB = 100000
EMBED_DIM = 64
BATCH = 4096
HIST = 50
_NB = BATCH // 128


def _make_gather():
    mesh = plsc.VectorSubcoreMesh(core_axis_name="c", subcore_axis_name="s")

    @functools.partial(
        pl.kernel,
        out_type=jax.ShapeDtypeStruct((HIST, 8, _NB, 8, 128), jnp.float32),
        mesh=mesh,
        scratch_types=[
            pltpu.VMEM((1, VOCAB), jnp.float32),
            pltpu.VMEM((BATCH,), jnp.int32),
            pltpu.VMEM((BATCH,), jnp.int32),
            pltpu.VMEM((_NB, 128), jnp.float32),
            pltpu.VMEM((_NB, 128), jnp.float32),
            pltpu.SemaphoreType.DMA,
            pltpu.SemaphoreType.DMA,
            pltpu.SemaphoreType.DMA,
        ],
        compiler_params=pltpu.CompilerParams(use_tc_tiling_on_sc=True,
                                             needs_layout_passes=False),
    )
    def gather_kernel(tableT_hbm, idx_hbm, out_hbm,
                      row_v, idx0_v, idx1_v, blk0_v, blk1_v,
                      isem, osem0, osem1):
        wid = lax.axis_index("s") * 2 + lax.axis_index("c")
        zero16 = jnp.zeros((16,), jnp.int32)
        idx_bufs = (idx0_v, idx1_v)
        blk_bufs = (blk0_v, blk1_v)
        osems = (osem0, osem1)

        def fill_blk(idx_v, blk_v):
            @plsc.parallel_loop(0, _NB, unroll=2)
            def _k(k):
                for s in range(8):
                    iv = idx_v[pl.ds(k * 128 + s * 16, 16)]
                    vals = plsc.load_gather(row_v, [zero16, iv])
                    blk_v[k, pl.ds(s * 16, 16)] = vals

        @pl.loop(0, 2)
        def _pass(p):
            e = p * 32 + wid
            e_hi = e // 8
            e_lo = e % 8
            pltpu.sync_copy(tableT_hbm.at[pl.ds(e, 1), :], row_v)
            pltpu.async_copy(idx_hbm.at[pl.ds(0, BATCH)], idx0_v, isem).wait()

            @pl.loop(0, HIST, step=2)
            def _h2(h):
                for s in range(2):
                    h_s = h + s
                    idx_v, blk_v = idx_bufs[s], blk_bufs[s]
                    nxt = idx_bufs[(s + 1) % 2]

                    @pl.when(h_s + 1 < HIST)
                    def _():
                        pltpu.async_copy(
                            idx_hbm.at[pl.ds((h_s + 1) * BATCH, BATCH)],
                            nxt, isem)

                    @pl.when(h_s >= 2)
                    def _():
                        pltpu.make_async_copy(
                            blk_v, out_hbm.at[0, 0, :, 0, :], osems[s]).wait()

                    fill_blk(idx_v, blk_v)
                    pltpu.async_copy(
                        blk_v, out_hbm.at[h_s, e_hi, :, e_lo, :], osems[s])

                    @pl.when(h_s + 1 < HIST)
                    def _():
                        pltpu.make_async_copy(
                            idx_hbm.at[pl.ds(0, BATCH)], nxt, isem).wait()

            for s in range(2):
                pltpu.make_async_copy(
                    blk_bufs[s], out_hbm.at[0, 0, :, 0, :], osems[s]).wait()

    return gather_kernel


_GATHER = _make_gather()


def kernel(vocab_embeddings, token_en):
    idx = token_en.T.reshape(BATCH * HIST).astype(jnp.int32)
    o5 = _GATHER(vocab_embeddings.T, idx)
    flat = o5.transpose(2, 4, 0, 1, 3).reshape(BATCH, HIST, EMBED_DIM)
    return (flat,)


# speedup vs baseline: 1.1588x; 1.1588x over previous
import functools

import jax
import jax.numpy as jnp
from jax import lax
from jax.experimental import pallas as pl
from jax.experimental.pallas import tpu as pltpu
from jax.experimental.pallas import tpu_sc as plsc

VOCAB = 100000
EMBED_DIM = 64
BATCH = 4096
HIST = 50
_NB = BATCH // 128


def _make_gather():
    mesh = plsc.VectorSubcoreMesh(core_axis_name="c", subcore_axis_name="s")

    @functools.partial(
        pl.kernel,
        out_type=jax.ShapeDtypeStruct((HIST, 8, _NB, 8, 128), jnp.float32),
        mesh=mesh,
        scratch_types=[
            pltpu.VMEM((1, VOCAB), jnp.float32),
            pltpu.VMEM((BATCH,), jnp.int32),
            pltpu.VMEM((BATCH,), jnp.int32),
            pltpu.VMEM((2, _NB, 128), jnp.float32),
            pltpu.VMEM((2, _NB, 128), jnp.float32),
            pltpu.SemaphoreType.DMA,
            pltpu.SemaphoreType.DMA,
            pltpu.SemaphoreType.DMA,
            pltpu.SemaphoreType.DMA,
        ],
        compiler_params=pltpu.CompilerParams(use_tc_tiling_on_sc=True,
                                             needs_layout_passes=False),
    )
    def gather_kernel(tableT_hbm, idx_hbm, out_hbm,
                      row_v, idx0_v, idx1_v, pblk0, pblk1,
                      isem0, isem1, osem0, osem1):
        wid = lax.axis_index("s") * 2 + lax.axis_index("c")
        zero16 = jnp.zeros((16,), jnp.int32)
        idxb = (idx0_v, idx1_v)
        isems = (isem0, isem1)
        pblks = (pblk0, pblk1)
        osems = (osem0, osem1)

        def fill_blk(idx_v, pblk, slot):
            @plsc.parallel_loop(0, _NB)
            def _k(k):
                for s in range(8):
                    iv = idx_v[pl.ds(k * 128 + s * 16, 16)]
                    vals = plsc.load_gather(row_v, [zero16, iv])
                    pblk[slot, k, pl.ds(s * 16, 16)] = vals

        def fire_idx(h, b):
            pltpu.async_copy(idx_hbm.at[pl.ds(h * BATCH, BATCH)],
                             idxb[b], isems[b])

        def wait_idx(b):
            pltpu.make_async_copy(idx_hbm.at[pl.ds(0, BATCH)],
                                  idxb[b], isems[b]).wait()

        def fire_out(h0, pb, e_hi, e_lo):
            pltpu.async_copy(pblks[pb],
                             out_hbm.at[pl.ds(h0, 2), e_hi, :, e_lo, :],
                             osems[pb])

        def wait_out(pb):
            pltpu.make_async_copy(pblks[pb],
                                  out_hbm.at[pl.ds(0, 2), 0, :, 0, :],
                                  osems[pb]).wait()

        def do_pair(g, pb, e_hi, e_lo, prefetch):
            h0 = 2 * g
            wait_idx(0)
            fill_blk(idx0_v, pblks[pb], 0)
            if prefetch:
                fire_idx(h0 + 2, 0)
            wait_idx(1)
            fill_blk(idx1_v, pblks[pb], 1)
            if prefetch:
                fire_idx(h0 + 3, 1)
            fire_out(h0, pb, e_hi, e_lo)

        @pl.loop(0, 2)
        def _pass(p):
            e = p * 32 + wid
            e_hi = e // 8
            e_lo = e % 8
            pltpu.sync_copy(tableT_hbm.at[pl.ds(e, 1), :], row_v)
            fire_idx(0, 0)
            fire_idx(1, 1)

            @pl.loop(0, 12)
            def _t(t):
                for pb in range(2):
                    g = 2 * t + pb

                    @pl.when(t >= 1)
                    def _():
                        wait_out(pb)

                    do_pair(g, pb, e_hi, e_lo, prefetch=True)

            wait_out(0)
            do_pair(24, 0, e_hi, e_lo, prefetch=False)
            wait_out(0)
            wait_out(1)

    return gather_kernel


_GATHER = _make_gather()


def kernel(vocab_embeddings, token_en):
    idx = token_en.T.reshape(BATCH * HIST).astype(jnp.int32)
    o5 = _GATHER(vocab_embeddings.T, idx)
    flat = o5.transpose(2, 4, 0, 1, 3).reshape(BATCH, HIST, EMBED_DIM)
    return (flat,)
